# Initial kernel scaffold; baseline (speedup 1.0000x reference)
import functools

import jax
import jax.numpy as jnp
from jax import lax
from jax.experimental import pallas as pl
from jax.experimental.pallas import tpu as pltpu
from jax.experimental.pallas import tpu_sc as plsc

N_USERS = 5000
N_NODES = 10000
N_EDGES = 160000
HALF = 128
N_LAYERS = 3
NS = 16
CH = 128
NCHUNK = 79
EPW = NCHUNK * CH
EPAD = NS * EPW
ACC_ROWS = 10240
DRAIN = N_NODES // NS


def _sc_body(emb_hbm, col_hbm, row_hbm, val_hbm, o1, o2, o3,
             col_v, row_v, val_v, buf, zero_v, acc, sem):
    cid = lax.axis_index("c")
    sid = lax.axis_index("s")

    pltpu.sync_copy(col_hbm.at[sid], col_v)
    pltpu.sync_copy(row_hbm.at[sid], row_v)
    pltpu.sync_copy(val_hbm.at[sid], val_v)

    z16 = jnp.zeros((16,), jnp.float32)

    @pl.loop(0, CH)
    def _(r):
        for g in range(8):
            zero_v[r, pl.ds(g * 16, 16)] = z16

    def do_layer(src, dst):
        for z in range(5):
            pltpu.sync_copy(zero_v, acc.at[pl.ds(sid * 640 + z * CH, CH)])
        plsc.subcore_barrier()

        @pl.loop(0, NCHUNK)
        def _(j):
            pltpu.async_copy(src.at[cid].at[col_v.at[j]], buf, sem).wait()

            @pl.loop(0, CH)
            def _(e):
                jv = jnp.full((16,), j, jnp.int32)
                ev = jnp.full((16,), e, jnp.int32)
                v = plsc.load_gather(val_v, [jv, ev])
                for g in range(8):
                    sl = pl.ds(g * 16, 16)
                    buf[e, sl] = buf[e, sl] * v

            pltpu.sync_copy(buf, acc.at[row_v.at[j]], add=True)

        plsc.subcore_barrier()
        pltpu.sync_copy(acc.at[pl.ds(sid * DRAIN, DRAIN)],
                        dst.at[cid].at[pl.ds(sid * DRAIN, DRAIN)])
        plsc.subcore_barrier()

    do_layer(emb_hbm, o1)
    do_layer(o1, o2)
    do_layer(o2, o3)


def _combine_body(e_ref, a_ref, b_ref, c_ref, o_ref):
    o_ref[...] = (e_ref[...] + a_ref[...] + b_ref[...] + c_ref[...]) * 0.25


def kernel(user_embedding, item_embedding, adj_indices, adj_values):
    f32 = jnp.float32
    all_emb = jnp.concatenate([user_embedding, item_embedding], axis=0)
    emb2 = all_emb.reshape(N_NODES, 2, HALF).transpose(1, 0, 2)

    pad = EPAD - N_EDGES
    row = jnp.concatenate([adj_indices[0], jnp.zeros((pad,), jnp.int32)])
    col = jnp.concatenate([adj_indices[1], jnp.zeros((pad,), jnp.int32)])
    val = jnp.concatenate([adj_values, jnp.zeros((pad,), f32)])
    rowp = row.reshape(NS, NCHUNK, CH)
    colp = col.reshape(NS, NCHUNK, CH)
    valp = val.reshape(NS, NCHUNK, CH)

    out_sds = jax.ShapeDtypeStruct((2, N_NODES, HALF), f32)
    sc_fn = pl.kernel(
        _sc_body,
        out_type=[out_sds, out_sds, out_sds],
        mesh=plsc.VectorSubcoreMesh(core_axis_name="c", subcore_axis_name="s"),
        scratch_types=[
            pltpu.VMEM((NCHUNK, CH), jnp.int32),
            pltpu.VMEM((NCHUNK, CH), jnp.int32),
            pltpu.VMEM((NCHUNK, CH), f32),
            pltpu.VMEM((CH, HALF), f32),
            pltpu.VMEM((CH, HALF), f32),
            pltpu.VMEM_SHARED((ACC_ROWS, HALF), f32),
            pltpu.SemaphoreType.DMA,
        ],
    )
    l1, l2, l3 = sc_fn(emb2, colp, rowp, valp)

    combined = pl.pallas_call(
        _combine_body,
        out_shape=jax.ShapeDtypeStruct((2, N_NODES, HALF), f32),
        grid=(2, 8),
        in_specs=[pl.BlockSpec((1, N_NODES // 8, HALF),
                               lambda i, j: (i, j, 0))] * 4,
        out_specs=pl.BlockSpec((1, N_NODES // 8, HALF), lambda i, j: (i, j, 0)),
    )(emb2, l1, l2, l3)

    final = combined.transpose(1, 0, 2).reshape(N_NODES, 2 * HALF)
    return (final[:N_USERS], final[N_USERS:])


# speedup vs baseline: 3.0020x; 3.0020x over previous
import dataclasses
import functools

import jax
import jax.numpy as jnp
from jax import lax
from jax.experimental import pallas as pl
from jax.experimental.pallas import tpu as pltpu
from jax.experimental.pallas import tpu_sc as plsc

N_USERS = 5000
N_NODES = 10000
N_EDGES = 160000
HALF = 128
N_LAYERS = 3
NS = 16
CH = 128
NCHUNK = 79
EPW = NCHUNK * CH
EPAD = NS * EPW
ACC_ROWS = 10240
DRAIN = ACC_ROWS // NS


def _sc_body(emb_hbm, col_hbm, row_hbm, val_hbm, o1, o2, o3,
             col_v, row_v, val_v, buf, acc, sem):
    cid = lax.axis_index("c")
    sid = lax.axis_index("s")

    pltpu.sync_copy(col_hbm.at[sid], col_v)
    pltpu.sync_copy(row_hbm.at[sid], row_v)
    pltpu.sync_copy(val_hbm.at[sid], val_v)

    z16 = jnp.zeros((16,), jnp.float32)

    def do_layer(src, dst):
        @pl.loop(0, CH)
        def _(r):
            for g in range(8):
                buf[r, pl.ds(g * 16, 16)] = z16

        for z in range(5):
            pltpu.sync_copy(buf, acc.at[pl.ds(sid * DRAIN + z * CH, CH)])
        plsc.subcore_barrier()

        @pl.loop(0, NCHUNK)
        def _(j):
            pltpu.async_copy(src.at[cid].at[col_v.at[j]], buf, sem).wait()

            @pl.loop(0, CH)
            def _(e):
                jv = jnp.full((16,), j, jnp.int32)
                ev = jnp.full((16,), e, jnp.int32)
                v = plsc.load_gather(val_v, [jv, ev])
                for g in range(8):
                    sl = pl.ds(g * 16, 16)
                    buf[e, sl] = buf[e, sl] * v

            pltpu.sync_copy(buf, acc.at[row_v.at[j]], add=True)

        plsc.subcore_barrier()
        pltpu.sync_copy(acc.at[pl.ds(sid * DRAIN, DRAIN)],
                        dst.at[cid].at[pl.ds(sid * DRAIN, DRAIN)])
        plsc.subcore_barrier()

    do_layer(emb_hbm, o1)
    do_layer(o1, o2)
    do_layer(o2, o3)


def _combine_body(e_ref, a_ref, b_ref, c_ref, o_ref):
    o_ref[...] = (e_ref[...] + a_ref[...] + b_ref[...] + c_ref[...]) * 0.25


def kernel(user_embedding, item_embedding, adj_indices, adj_values):
    f32 = jnp.float32
    all_emb = jnp.concatenate([user_embedding, item_embedding], axis=0)
    emb2 = all_emb.reshape(N_NODES, 2, HALF).transpose(1, 0, 2)
    emb2 = jnp.pad(emb2, ((0, 0), (0, ACC_ROWS - N_NODES), (0, 0)))

    pad = EPAD - N_EDGES
    row = jnp.concatenate([adj_indices[0], jnp.zeros((pad,), jnp.int32)])
    col = jnp.concatenate([adj_indices[1], jnp.zeros((pad,), jnp.int32)])
    val = jnp.concatenate([adj_values, jnp.zeros((pad,), f32)])
    rowp = row.reshape(NS, NCHUNK, CH)
    colp = col.reshape(NS, NCHUNK, CH)
    valp = val.reshape(NS, NCHUNK, CH)

    cp = pltpu.CompilerParams()
    if "needs_layout_passes" in pltpu.CompilerParams.__dataclass_fields__:
        cp = dataclasses.replace(cp, needs_layout_passes=False)

    out_sds = jax.ShapeDtypeStruct((2, ACC_ROWS, HALF), f32)
    sc_fn = pl.kernel(
        _sc_body,
        out_type=[out_sds, out_sds, out_sds],
        mesh=plsc.VectorSubcoreMesh(core_axis_name="c", subcore_axis_name="s"),
        scratch_types=[
            pltpu.VMEM((NCHUNK, CH), jnp.int32),
            pltpu.VMEM((NCHUNK, CH), jnp.int32),
            pltpu.VMEM((NCHUNK, CH), f32),
            pltpu.VMEM((CH, HALF), f32),
            pltpu.VMEM_SHARED((ACC_ROWS, HALF), f32),
            pltpu.SemaphoreType.DMA,
        ],
        compiler_params=cp,
    )
    l1, l2, l3 = sc_fn(emb2, colp, rowp, valp)

    combined = pl.pallas_call(
        _combine_body,
        out_shape=jax.ShapeDtypeStruct((2, ACC_ROWS, HALF), f32),
        grid=(2, 10),
        in_specs=[pl.BlockSpec((1, ACC_ROWS // 10, HALF),
                               lambda i, j: (i, j, 0))] * 4,
        out_specs=pl.BlockSpec((1, ACC_ROWS // 10, HALF), lambda i, j: (i, j, 0)),
    )(emb2, l1, l2, l3)

    final = combined[:, :N_NODES, :].transpose(1, 0, 2).reshape(N_NODES, 2 * HALF)
    return (final[:N_USERS], final[N_USERS:])
